# Initial kernel scaffold; baseline (speedup 1.0000x reference)
import jax
import jax.numpy as jnp
from jax.experimental import pallas as pl


def kernel(target_x, target_edge_index, target_batch, W1, b1, W2, b2, W3, b3, Wf1, bf1, Wf2, bf2):
    raise NotImplementedError("write your pallas kernel here")


# speedup vs baseline: 2.6295x; 2.6295x over previous
import jax
import jax.numpy as jnp
from jax.experimental import pallas as pl

_NB = 1000


def _matmul_scale_block(x_ref, w_ref, d_ref, o_ref):
    o_ref[...] = jnp.dot(x_ref[...], w_ref[...],
                         preferred_element_type=jnp.float32) * d_ref[...]


def _matmul_scale(x, W, d):
    n, f = x.shape
    fo = W.shape[1]
    return pl.pallas_call(
        _matmul_scale_block,
        grid=(n // _NB,),
        in_specs=[
            pl.BlockSpec((_NB, f), lambda i: (i, 0)),
            pl.BlockSpec((f, fo), lambda i: (0, 0)),
            pl.BlockSpec((_NB, 1), lambda i: (i, 0)),
        ],
        out_specs=pl.BlockSpec((_NB, fo), lambda i: (i, 0)),
        out_shape=jax.ShapeDtypeStruct((n, fo), jnp.float32),
    )(x, W, d)


def _finish_block(s_ref, hs_ref, d_ref, b_ref, o_ref):
    o_ref[...] = jnp.maximum(
        (s_ref[...] + hs_ref[...]) * d_ref[...] + b_ref[...], 0.0)


def _finish(S, hs, d, b):
    n, fo = S.shape
    return pl.pallas_call(
        _finish_block,
        grid=(n // _NB,),
        in_specs=[
            pl.BlockSpec((_NB, fo), lambda i: (i, 0)),
            pl.BlockSpec((_NB, fo), lambda i: (i, 0)),
            pl.BlockSpec((_NB, 1), lambda i: (i, 0)),
            pl.BlockSpec((1, fo), lambda i: (0, 0)),
        ],
        out_specs=pl.BlockSpec((_NB, fo), lambda i: (i, 0)),
        out_shape=jax.ShapeDtypeStruct((n, fo), jnp.float32),
    )(S, hs, d, b)


def _pool_block(oh_ref, h_ref, acc_ref):
    @pl.when(pl.program_id(0) == 0)
    def _():
        acc_ref[...] = jnp.zeros_like(acc_ref)
    acc_ref[...] += jax.lax.dot_general(
        oh_ref[...], h_ref[...], (((0,), (0,)), ((), ())),
        preferred_element_type=jnp.float32)


def _pool(onehot, hcat):
    n, g = onehot.shape
    fc = hcat.shape[1]
    return pl.pallas_call(
        _pool_block,
        grid=(n // _NB,),
        in_specs=[
            pl.BlockSpec((_NB, g), lambda i: (i, 0)),
            pl.BlockSpec((_NB, fc), lambda i: (i, 0)),
        ],
        out_specs=pl.BlockSpec((g, fc), lambda i: (0, 0)),
        out_shape=jax.ShapeDtypeStruct((g, fc), jnp.float32),
    )(onehot, hcat)


def _head_block(p_ref, wf1_ref, bf1_ref, wf2_ref, bf2_ref, o_ref):
    sums = p_ref[:, :-1]
    counts = jnp.maximum(p_ref[:, -1:], 1.0)
    mean = sums / counts
    r = jnp.maximum(jnp.dot(mean, wf1_ref[...],
                            preferred_element_type=jnp.float32)
                    + bf1_ref[...], 0.0)
    o_ref[...] = jnp.dot(r, wf2_ref[...],
                         preferred_element_type=jnp.float32) + bf2_ref[...]


def _head(pooled, Wf1, bf1, Wf2, bf2):
    g, fc = pooled.shape
    k, m = Wf1.shape
    mo = Wf2.shape[1]
    return pl.pallas_call(
        _head_block,
        in_specs=[
            pl.BlockSpec((g, fc), lambda: (0, 0)),
            pl.BlockSpec((k, m), lambda: (0, 0)),
            pl.BlockSpec((1, m), lambda: (0, 0)),
            pl.BlockSpec((m, mo), lambda: (0, 0)),
            pl.BlockSpec((1, mo), lambda: (0, 0)),
        ],
        out_specs=pl.BlockSpec((g, mo), lambda: (0, 0)),
        out_shape=jax.ShapeDtypeStruct((g, mo), jnp.float32),
    )(pooled, Wf1, bf1, Wf2, bf2)


def kernel(target_x, target_edge_index, target_batch,
           W1, b1, W2, b2, W3, b3, Wf1, bf1, Wf2, bf2):
    n = target_x.shape[0]
    g = 64
    row = target_edge_index[0]
    col = target_edge_index[1]

    deg = jnp.ones((n,), jnp.float32).at[col].add(1.0)
    dinv = deg ** -0.5
    d = dinv[:, None]

    h = target_x
    for W, b in ((W1, b1), (W2, b2), (W3, b3)):
        hs = _matmul_scale(h, W, d)
        S = jnp.zeros_like(hs).at[col].add(hs[row])
        h = _finish(S, hs, d, b.reshape(1, -1))

    onehot = (target_batch[:, None] ==
              jnp.arange(g, dtype=target_batch.dtype)[None, :]
              ).astype(jnp.float32)
    hcat = jnp.concatenate([h, jnp.ones((n, 1), jnp.float32)], axis=1)
    pooled = _pool(onehot, hcat)
    return _head(pooled, Wf1, bf1.reshape(1, -1), Wf2, bf2.reshape(1, -1))
